# baseline (speedup 1.0000x reference)
import functools
import jax
import jax.numpy as jnp
import numpy as np
from jax import lax
from jax.experimental import pallas as pl
from jax.experimental.pallas import tpu as pltpu, tpu_sc as plsc

_B, _T, _N, _FIN = 1, 8, 10000, 1
_E = 160000
_H = 128
_NH = 8
_DH = _H // _NH
_HOR = 8

_NP = 10240
_C = 4000
_EHALF = _E // 2
_NCH = _EHALF // _C
_STEPS = _C // 16
_NB = 1024
_NCORES = 2
_NSUB = 16


def _sc_edge_body(x_hbm, src_hbm, dst_hbm, ew_hbm, par_hbm, o_hbm,
                  x_v, src_v0, src_v1, dst_v0, dst_v1, ew_v0, ew_v1, par_v, tab_v,
                  sem_s0, sem_d0, sem_w0, sem_s1, sem_d1, sem_w1):
    cid = lax.axis_index("c")
    sid = lax.axis_index("s")
    wid = sid * _NCORES + cid
    hh = wid // 16
    rem = wid - hh * 16
    eh = rem // 8
    t = rem - eh * 8
    sems = ((sem_s0, sem_d0, sem_w0), (sem_s1, sem_d1, sem_w1))
    bufs = ((src_v0, dst_v0, ew_v0), (src_v1, dst_v1, ew_v1))

    e0 = eh * _EHALF

    def start(ci, b):
        off = e0 + ci * _C
        pltpu.async_copy(src_hbm.at[pl.ds(off, _C)], bufs[b][0], sems[b][0])
        pltpu.async_copy(dst_hbm.at[pl.ds(off, _C)], bufs[b][1], sems[b][1])
        pltpu.async_copy(ew_hbm.at[pl.ds(off, _C)], bufs[b][2], sems[b][2])

    def wait(b):
        pltpu.make_async_copy(src_hbm.at[pl.ds(0, _C)], bufs[b][0], sems[b][0]).wait()
        pltpu.make_async_copy(dst_hbm.at[pl.ds(0, _C)], bufs[b][1], sems[b][1]).wait()
        pltpu.make_async_copy(ew_hbm.at[pl.ds(0, _C)], bufs[b][2], sems[b][2]).wait()

    start(0, 0)
    start(1, 1)
    pltpu.sync_copy(x_hbm.at[t], x_v)
    pltpu.sync_copy(par_hbm.at[pl.ds((t * 2 + hh) * 16, 16)], par_v)

    zeros = jnp.zeros((16,), jnp.float32)
    lanes = lax.iota(jnp.int32, 16)

    @plsc.parallel_loop(0, (8 * _NP) // 16, unroll=8)
    def _zero(i):
        tab_v[pl.ds(i * 16, 16)] = zeros

    def pair_body(pi, carry):
        for b in range(2):
            ci = pi * 2 + b
            wait(b)

            @plsc.parallel_loop(0, _STEPS, unroll=4)
            def _step(s):
                sv = bufs[b][0][pl.ds(s * 16, 16)]
                dv = bufs[b][1][pl.ds(s * 16, 16)]
                ewv = bufs[b][2][pl.ds(s * 16, 16)]
                xs = plsc.load_gather(x_v, [sv])
                xd = plsc.load_gather(x_v, [dv])
                for j in range(4):
                    asv = par_v[j * 4 + 0]
                    adv = par_v[j * 4 + 1]
                    bqv = par_v[j * 4 + 2]
                    mmv = par_v[j * 4 + 3]
                    z = xs * asv + xd * adv + bqv
                    zl = jnp.maximum(z, 0.2 * z)
                    w = jnp.exp(zl - mmv) * ewv
                    plsc.addupdate_scatter(tab_v, [dv + (j * _NP)], w)
                    plsc.addupdate_scatter(tab_v, [dv + ((4 + j) * _NP)], w * xs)

            @pl.when(ci + 2 < _NCH)
            def _():
                start(ci + 2, b)
        return carry

    lax.fori_loop(0, _NCH // 2, pair_body, 0)
    pltpu.sync_copy(tab_v, o_hbm.at[wid])


def _sc_edge_pass(x2p, src, dst, ew, par):
    mesh = plsc.VectorSubcoreMesh(core_axis_name="c", subcore_axis_name="s")
    f = pl.kernel(
        _sc_edge_body,
        out_type=jax.ShapeDtypeStruct((32, 8 * _NP), jnp.float32),
        mesh=mesh,
        scratch_types=[
            pltpu.VMEM((_NP,), jnp.float32),
            pltpu.VMEM((_C,), jnp.int32),
            pltpu.VMEM((_C,), jnp.int32),
            pltpu.VMEM((_C,), jnp.int32),
            pltpu.VMEM((_C,), jnp.int32),
            pltpu.VMEM((_C,), jnp.float32),
            pltpu.VMEM((_C,), jnp.float32),
            pltpu.VMEM((16, 16), jnp.float32),
            pltpu.VMEM((8 * _NP,), jnp.float32),
            pltpu.SemaphoreType.DMA,
            pltpu.SemaphoreType.DMA,
            pltpu.SemaphoreType.DMA,
            pltpu.SemaphoreType.DMA,
            pltpu.SemaphoreType.DMA,
            pltpu.SemaphoreType.DMA,
        ],
        compiler_params=pltpu.CompilerParams(needs_layout_passes=False),
    )
    return f(x2p, src, dst, ew, par)


def _tc_tail_body(o_ref, x_ref, m_ref, w3_ref, u_ref, cst_ref, bo_ref, out_ref):
    ob = o_ref[...]
    o5 = ob.reshape(2, 2, _T, 8, _NB)
    acc = o5[:, 0] + o5[:, 1]
    den = acc[:, :, 0:4, :]
    num = acc[:, :, 4:8, :]
    bo = bo_ref[...]
    pred = lax.dot_general(u_ref[...], x_ref[...],
                           (((0,), (0,)), ((), ())),
                           preferred_element_type=jnp.float32)
    for t in range(_T):
        d_t = den[:, t].reshape(_NH, _NB)
        n_t = num[:, t].reshape(_NH, _NB)
        inv = 1.0 / (d_t + 1e-16)
        r_t = n_t * inv
        s_t = d_t * inv
        rs = jnp.concatenate([r_t, s_t], axis=0)
        ppre = lax.dot_general(m_ref[t], rs,
                               (((0,), (0,)), ((), ())),
                               preferred_element_type=jnp.float32)
        p = jnp.maximum(ppre + bo, 0.0)
        pred = pred + lax.dot_general(w3_ref[t], p,
                                      (((0,), (0,)), ((), ())),
                                      preferred_element_type=jnp.float32)
    out_ref[...] = pred + cst_ref[...]


def _tc_tail(o3, x2p, m, w3, u, cst, bo):
    grid = (_NP // _NB,)
    return pl.pallas_call(
        _tc_tail_body,
        out_shape=jax.ShapeDtypeStruct((_HOR, _NP), jnp.float32),
        grid=grid,
        in_specs=[
            pl.BlockSpec((32, _T, _NB), lambda i: (0, 0, i)),
            pl.BlockSpec((_T, _NB), lambda i: (0, i)),
            pl.BlockSpec((_T, 16, _H), lambda i: (0, 0, 0)),
            pl.BlockSpec((_T, _H, _HOR), lambda i: (0, 0, 0)),
            pl.BlockSpec((_T, _HOR), lambda i: (0, 0)),
            pl.BlockSpec((_HOR, 1), lambda i: (0, 0)),
            pl.BlockSpec((_H, 1), lambda i: (0, 0)),
        ],
        out_specs=pl.BlockSpec((_HOR, _NB), lambda i: (0, i)),
    )(o3, x2p, m, w3, u, cst, bo)


def _sinusoidal_pe(positions, d):
    pos = positions[:, None].astype(jnp.float32)
    i = jnp.arange(d // 2, dtype=jnp.float32)[None, :]
    angles = pos / jnp.power(10000.0, 2.0 * i / d)
    return jnp.concatenate([jnp.sin(angles), jnp.cos(angles)], axis=-1)


def kernel(x, edge_index, edge_weight, fq_param, W_enc, b_enc, W_gat,
           a_src, a_dst, W_out, b_out, W_dec, b_dec):
    x2 = x[0, :, :, 0]
    src = edge_index[0]
    dst = edge_index[1]

    pe = _sinusoidal_pe(jnp.arange(_T), _H)
    gv = W_enc[0] @ W_gat
    c = (b_enc[None, :] + pe) @ W_gat
    gh = gv.reshape(_NH, _DH)
    ch = c.reshape(_T, _NH, _DH)
    As = (gh * a_src).sum(-1)
    Ad = (gh * a_dst).sum(-1)
    Bq = (ch * a_src).sum(-1) + (ch * a_dst).sum(-1)
    Mx = x2.max(1)
    mx = x2.min(1)
    zmax = (jnp.where(As[None, :] > 0, As[None, :] * Mx[:, None], As[None, :] * mx[:, None])
            + jnp.where(Ad[None, :] > 0, Ad[None, :] * Mx[:, None], Ad[None, :] * mx[:, None])
            + Bq)
    mM = jnp.maximum(zmax, 0.2 * zmax)

    Wo3 = W_out.reshape(_NH, _DH, _H)
    G2 = jnp.einsum('hd,hdo->ho', gh, Wo3)
    C2 = jnp.einsum('thd,hdo->tho', ch, Wo3)
    W3 = W_dec.reshape(_T, _H, _HOR * _FIN)
    u = jnp.einsum('k,tko->to', W_enc[0], W3)
    cst = jnp.einsum('tk,tko->o', b_enc[None, :] + pe, W3) + b_dec

    stacked = jnp.stack([
        jnp.broadcast_to(As[None, :], (_T, _NH)), jnp.broadcast_to(Ad[None, :], (_T, _NH)),
        Bq, mM], axis=-1)
    par = jnp.broadcast_to(
        stacked.reshape(_T, 2, 4, 4)[..., None], (_T, 2, 4, 4, 16)
    ).reshape(_T * 2 * 16, 16)

    x2p = jnp.zeros((_T, _NP), jnp.float32).at[:, :_N].set(x2)

    o = _sc_edge_pass(x2p, src, dst, edge_weight, par)
    o3 = o.reshape(32, 8, _NP)

    m = jnp.concatenate([jnp.broadcast_to(G2[None], (_T, _NH, _H)), C2], axis=1)

    pred = _tc_tail(o3, x2p, m, W3, u, cst.reshape(_HOR, 1), b_out.reshape(_H, 1))
    return pred[:, :_N].reshape(1, _HOR, _N, _FIN)


# speedup vs baseline: 1.0245x; 1.0245x over previous
import functools
import jax
import jax.numpy as jnp
import numpy as np
from jax import lax
from jax.experimental import pallas as pl
from jax.experimental.pallas import tpu as pltpu, tpu_sc as plsc

_B, _T, _N, _FIN = 1, 8, 10000, 1
_E = 160000
_H = 128
_NH = 8
_DH = _H // _NH
_HOR = 8

_NP = 10240
_C = 2000
_EHALF = _E // 2
_NCH = _EHALF // _C
_STEPS = _C // 16
_NB = 1024
_NCORES = 2
_NSUB = 16


def _sc_edge_body(x_hbm, src_hbm, dst_hbm, ew_hbm, par_hbm, o_hbm,
                  x_v, src_v0, src_v1, dst_v0, dst_v1, ew_v0, ew_v1, par_v, tab_v,
                  sem_s0, sem_d0, sem_w0, sem_s1, sem_d1, sem_w1):
    cid = lax.axis_index("c")
    sid = lax.axis_index("s")
    wid = sid * _NCORES + cid
    hh = wid // 16
    rem = wid - hh * 16
    eh = rem // 8
    t = rem - eh * 8
    sems = ((sem_s0, sem_d0, sem_w0), (sem_s1, sem_d1, sem_w1))
    bufs = ((src_v0, dst_v0, ew_v0), (src_v1, dst_v1, ew_v1))

    e0 = eh * _EHALF

    def start(ci, b):
        off = e0 + ci * _C
        pltpu.async_copy(src_hbm.at[pl.ds(off, _C)], bufs[b][0], sems[b][0])
        pltpu.async_copy(dst_hbm.at[pl.ds(off, _C)], bufs[b][1], sems[b][1])
        pltpu.async_copy(ew_hbm.at[pl.ds(off, _C)], bufs[b][2], sems[b][2])

    def wait(b):
        pltpu.make_async_copy(src_hbm.at[pl.ds(0, _C)], bufs[b][0], sems[b][0]).wait()
        pltpu.make_async_copy(dst_hbm.at[pl.ds(0, _C)], bufs[b][1], sems[b][1]).wait()
        pltpu.make_async_copy(ew_hbm.at[pl.ds(0, _C)], bufs[b][2], sems[b][2]).wait()

    start(0, 0)
    start(1, 1)
    pltpu.sync_copy(x_hbm.at[t], x_v)
    pltpu.sync_copy(par_hbm.at[pl.ds((t * 2 + hh) * 16, 16)], par_v)

    zeros = jnp.zeros((16,), jnp.float32)
    lanes = lax.iota(jnp.int32, 16)

    @plsc.parallel_loop(0, (8 * _NP) // 16, unroll=8)
    def _zero(i):
        tab_v[pl.ds(i * 16, 16)] = zeros

    def pair_body(pi, carry):
        for b in range(2):
            ci = pi * 2 + b
            wait(b)

            @plsc.parallel_loop(0, _STEPS, unroll=4)
            def _step(s):
                sv = bufs[b][0][pl.ds(s * 16, 16)]
                dv = bufs[b][1][pl.ds(s * 16, 16)]
                ewv = bufs[b][2][pl.ds(s * 16, 16)]
                xs = plsc.load_gather(x_v, [sv])
                xd = plsc.load_gather(x_v, [dv])
                for j in range(4):
                    asv = par_v[j * 4 + 0]
                    adv = par_v[j * 4 + 1]
                    bqv = par_v[j * 4 + 2]
                    mmv = par_v[j * 4 + 3]
                    z = xs * asv + xd * adv + bqv
                    zl = jnp.maximum(z, 0.2 * z)
                    w = jnp.exp(zl - mmv) * ewv
                    plsc.addupdate_scatter(tab_v, [dv + (j * _NP)], w)
                    plsc.addupdate_scatter(tab_v, [dv + ((4 + j) * _NP)], w * xs)

            @pl.when(ci + 2 < _NCH)
            def _():
                start(ci + 2, b)
        return carry

    lax.fori_loop(0, _NCH // 2, pair_body, 0)
    pltpu.sync_copy(tab_v, o_hbm.at[wid])


def _sc_edge_pass(x2p, src, dst, ew, par):
    mesh = plsc.VectorSubcoreMesh(core_axis_name="c", subcore_axis_name="s")
    f = pl.kernel(
        _sc_edge_body,
        out_type=jax.ShapeDtypeStruct((32, 8 * _NP), jnp.float32),
        mesh=mesh,
        scratch_types=[
            pltpu.VMEM((_NP,), jnp.float32),
            pltpu.VMEM((_C,), jnp.int32),
            pltpu.VMEM((_C,), jnp.int32),
            pltpu.VMEM((_C,), jnp.int32),
            pltpu.VMEM((_C,), jnp.int32),
            pltpu.VMEM((_C,), jnp.float32),
            pltpu.VMEM((_C,), jnp.float32),
            pltpu.VMEM((16, 16), jnp.float32),
            pltpu.VMEM((8 * _NP,), jnp.float32),
            pltpu.SemaphoreType.DMA,
            pltpu.SemaphoreType.DMA,
            pltpu.SemaphoreType.DMA,
            pltpu.SemaphoreType.DMA,
            pltpu.SemaphoreType.DMA,
            pltpu.SemaphoreType.DMA,
        ],
        compiler_params=pltpu.CompilerParams(needs_layout_passes=False),
    )
    return f(x2p, src, dst, ew, par)


def _tc_tail_body(o_ref, x_ref, m_ref, w3_ref, u_ref, cst_ref, bo_ref, out_ref):
    ob = o_ref[...]
    o5 = ob.reshape(2, 2, _T, 8, _NB)
    acc = o5[:, 0] + o5[:, 1]
    den = acc[:, :, 0:4, :]
    num = acc[:, :, 4:8, :]
    bo = bo_ref[...]
    pred = lax.dot_general(u_ref[...], x_ref[...],
                           (((0,), (0,)), ((), ())),
                           preferred_element_type=jnp.float32)
    for t in range(_T):
        d_t = den[:, t].reshape(_NH, _NB)
        n_t = num[:, t].reshape(_NH, _NB)
        inv = 1.0 / (d_t + 1e-16)
        r_t = n_t * inv
        s_t = d_t * inv
        rs = jnp.concatenate([r_t, s_t], axis=0)
        ppre = lax.dot_general(m_ref[t], rs,
                               (((0,), (0,)), ((), ())),
                               preferred_element_type=jnp.float32)
        p = jnp.maximum(ppre + bo, 0.0)
        pred = pred + lax.dot_general(w3_ref[t], p,
                                      (((0,), (0,)), ((), ())),
                                      preferred_element_type=jnp.float32)
    out_ref[...] = pred + cst_ref[...]


def _tc_tail(o3, x2p, m, w3, u, cst, bo):
    grid = (_NP // _NB,)
    return pl.pallas_call(
        _tc_tail_body,
        out_shape=jax.ShapeDtypeStruct((_HOR, _NP), jnp.float32),
        grid=grid,
        in_specs=[
            pl.BlockSpec((32, _T, _NB), lambda i: (0, 0, i)),
            pl.BlockSpec((_T, _NB), lambda i: (0, i)),
            pl.BlockSpec((_T, 16, _H), lambda i: (0, 0, 0)),
            pl.BlockSpec((_T, _H, _HOR), lambda i: (0, 0, 0)),
            pl.BlockSpec((_T, _HOR), lambda i: (0, 0)),
            pl.BlockSpec((_HOR, 1), lambda i: (0, 0)),
            pl.BlockSpec((_H, 1), lambda i: (0, 0)),
        ],
        out_specs=pl.BlockSpec((_HOR, _NB), lambda i: (0, i)),
    )(o3, x2p, m, w3, u, cst, bo)


def _sinusoidal_pe(positions, d):
    pos = positions[:, None].astype(jnp.float32)
    i = jnp.arange(d // 2, dtype=jnp.float32)[None, :]
    angles = pos / jnp.power(10000.0, 2.0 * i / d)
    return jnp.concatenate([jnp.sin(angles), jnp.cos(angles)], axis=-1)


def kernel(x, edge_index, edge_weight, fq_param, W_enc, b_enc, W_gat,
           a_src, a_dst, W_out, b_out, W_dec, b_dec):
    x2 = x[0, :, :, 0]
    src = edge_index[0]
    dst = edge_index[1]

    pe = _sinusoidal_pe(jnp.arange(_T), _H)
    gv = W_enc[0] @ W_gat
    c = (b_enc[None, :] + pe) @ W_gat
    gh = gv.reshape(_NH, _DH)
    ch = c.reshape(_T, _NH, _DH)
    As = (gh * a_src).sum(-1)
    Ad = (gh * a_dst).sum(-1)
    Bq = (ch * a_src).sum(-1) + (ch * a_dst).sum(-1)
    Mx = x2.max(1)
    mx = x2.min(1)
    zmax = (jnp.where(As[None, :] > 0, As[None, :] * Mx[:, None], As[None, :] * mx[:, None])
            + jnp.where(Ad[None, :] > 0, Ad[None, :] * Mx[:, None], Ad[None, :] * mx[:, None])
            + Bq)
    mM = jnp.maximum(zmax, 0.2 * zmax)

    Wo3 = W_out.reshape(_NH, _DH, _H)
    G2 = jnp.einsum('hd,hdo->ho', gh, Wo3)
    C2 = jnp.einsum('thd,hdo->tho', ch, Wo3)
    W3 = W_dec.reshape(_T, _H, _HOR * _FIN)
    u = jnp.einsum('k,tko->to', W_enc[0], W3)
    cst = jnp.einsum('tk,tko->o', b_enc[None, :] + pe, W3) + b_dec

    stacked = jnp.stack([
        jnp.broadcast_to(As[None, :], (_T, _NH)), jnp.broadcast_to(Ad[None, :], (_T, _NH)),
        Bq, mM], axis=-1)
    par = jnp.broadcast_to(
        stacked.reshape(_T, 2, 4, 4)[..., None], (_T, 2, 4, 4, 16)
    ).reshape(_T * 2 * 16, 16)

    x2p = jnp.zeros((_T, _NP), jnp.float32).at[:, :_N].set(x2)

    o = _sc_edge_pass(x2p, src, dst, edge_weight, par)
    o3 = o.reshape(32, 8, _NP)

    m = jnp.concatenate([jnp.broadcast_to(G2[None], (_T, _NH, _H)), C2], axis=1)

    pred = _tc_tail(o3, x2p, m, W3, u, cst.reshape(_HOR, 1), b_out.reshape(_H, 1))
    return pred[:, :_N].reshape(1, _HOR, _N, _FIN)


# speedup vs baseline: 1.0787x; 1.0529x over previous
import functools
import jax
import jax.numpy as jnp
import numpy as np
from jax import lax
from jax.experimental import pallas as pl
from jax.experimental.pallas import tpu as pltpu, tpu_sc as plsc

_B, _T, _N, _FIN = 1, 8, 10000, 1
_E = 160000
_H = 128
_NH = 8
_DH = _H // _NH
_HOR = 8

_NP = 10000
_C = 2000
_EHALF = _E // 2
_NCH = _EHALF // _C
_STEPS = _C // 16
_NB = _N
_NCORES = 2
_NSUB = 16


def _sc_edge_body(x_hbm, src_hbm, dst_hbm, ew_hbm, par_hbm, o_hbm,
                  x_v, src_v0, src_v1, dst_v0, dst_v1, ew_v0, ew_v1, par_v,
                  t0, t1, t2, t3, t4, t5, t6, t7,
                  sem_s0, sem_d0, sem_w0, sem_s1, sem_d1, sem_w1):
    tabs = (t0, t1, t2, t3, t4, t5, t6, t7)
    cid = lax.axis_index("c")
    sid = lax.axis_index("s")
    wid = sid * _NCORES + cid
    hh = wid // 16
    rem = wid - hh * 16
    eh = rem // 8
    t = rem - eh * 8
    sems = ((sem_s0, sem_d0, sem_w0), (sem_s1, sem_d1, sem_w1))
    bufs = ((src_v0, dst_v0, ew_v0), (src_v1, dst_v1, ew_v1))

    e0 = eh * _EHALF

    def start(ci, b):
        off = e0 + ci * _C
        pltpu.async_copy(src_hbm.at[pl.ds(off, _C)], bufs[b][0], sems[b][0])
        pltpu.async_copy(dst_hbm.at[pl.ds(off, _C)], bufs[b][1], sems[b][1])
        pltpu.async_copy(ew_hbm.at[pl.ds(off, _C)], bufs[b][2], sems[b][2])

    def wait(b):
        pltpu.make_async_copy(src_hbm.at[pl.ds(0, _C)], bufs[b][0], sems[b][0]).wait()
        pltpu.make_async_copy(dst_hbm.at[pl.ds(0, _C)], bufs[b][1], sems[b][1]).wait()
        pltpu.make_async_copy(ew_hbm.at[pl.ds(0, _C)], bufs[b][2], sems[b][2]).wait()

    start(0, 0)
    start(1, 1)
    pltpu.sync_copy(x_hbm.at[pl.ds(t * _NP, _NP)], x_v)
    pltpu.sync_copy(par_hbm.at[pl.ds((t * 2 + hh) * 16, 16)], par_v)

    zeros = jnp.zeros((16,), jnp.float32)
    lanes = lax.iota(jnp.int32, 16)

    @plsc.parallel_loop(0, _NP // 16, unroll=8)
    def _zero(i):
        for k in range(8):
            tabs[k][pl.ds(i * 16, 16)] = zeros

    def pair_body(pi, carry):
        for b in range(2):
            ci = pi * 2 + b
            wait(b)

            @plsc.parallel_loop(0, _STEPS, unroll=4)
            def _step(s):
                sv = bufs[b][0][pl.ds(s * 16, 16)]
                dv = bufs[b][1][pl.ds(s * 16, 16)]
                ewv = bufs[b][2][pl.ds(s * 16, 16)]
                xs = plsc.load_gather(x_v, [sv])
                xd = plsc.load_gather(x_v, [dv])
                for j in range(4):
                    asv = par_v[j * 4 + 0]
                    adv = par_v[j * 4 + 1]
                    bqv = par_v[j * 4 + 2]
                    mmv = par_v[j * 4 + 3]
                    z = xs * asv + xd * adv + bqv
                    zl = jnp.maximum(z, 0.2 * z)
                    w = jnp.exp(zl - mmv) * ewv
                    plsc.addupdate_scatter(tabs[j], [dv], w)
                    plsc.addupdate_scatter(tabs[4 + j], [dv], w * xs)

            @pl.when(ci + 2 < _NCH)
            def _():
                start(ci + 2, b)
        return carry

    lax.fori_loop(0, _NCH // 2, pair_body, 0)
    base = wid * (8 * _NP)
    for k in range(8):
        pltpu.sync_copy(tabs[k], o_hbm.at[pl.ds(base + k * _NP, _NP)])


def _sc_edge_pass(x2, src, dst, ew, par):
    mesh = plsc.VectorSubcoreMesh(core_axis_name="c", subcore_axis_name="s")
    f = pl.kernel(
        _sc_edge_body,
        out_type=jax.ShapeDtypeStruct((32 * 8 * _NP,), jnp.float32),
        mesh=mesh,
        scratch_types=[
            pltpu.VMEM((_NP,), jnp.float32),
            pltpu.VMEM((_C,), jnp.int32),
            pltpu.VMEM((_C,), jnp.int32),
            pltpu.VMEM((_C,), jnp.int32),
            pltpu.VMEM((_C,), jnp.int32),
            pltpu.VMEM((_C,), jnp.float32),
            pltpu.VMEM((_C,), jnp.float32),
            pltpu.VMEM((16, 16), jnp.float32),
            pltpu.VMEM((_NP,), jnp.float32),
            pltpu.VMEM((_NP,), jnp.float32),
            pltpu.VMEM((_NP,), jnp.float32),
            pltpu.VMEM((_NP,), jnp.float32),
            pltpu.VMEM((_NP,), jnp.float32),
            pltpu.VMEM((_NP,), jnp.float32),
            pltpu.VMEM((_NP,), jnp.float32),
            pltpu.VMEM((_NP,), jnp.float32),
            pltpu.SemaphoreType.DMA,
            pltpu.SemaphoreType.DMA,
            pltpu.SemaphoreType.DMA,
            pltpu.SemaphoreType.DMA,
            pltpu.SemaphoreType.DMA,
            pltpu.SemaphoreType.DMA,
        ],
        compiler_params=pltpu.CompilerParams(needs_layout_passes=False),
    )
    return f(x2.reshape(-1), src, dst, ew, par)


def _tc_tail_body(o_ref, x_ref, m_ref, w3_ref, u_ref, cst_ref, bo_ref, out_ref):
    ob = o_ref[...]
    o5 = ob.reshape(2, 2, _T, 8, _NB)
    acc = o5[:, 0] + o5[:, 1]
    den = acc[:, :, 0:4, :]
    num = acc[:, :, 4:8, :]
    bo = bo_ref[...]
    pred = lax.dot_general(u_ref[...], x_ref[...],
                           (((0,), (0,)), ((), ())),
                           preferred_element_type=jnp.float32)
    for t in range(_T):
        d_t = den[:, t].reshape(_NH, _NB)
        n_t = num[:, t].reshape(_NH, _NB)
        inv = 1.0 / (d_t + 1e-16)
        r_t = n_t * inv
        s_t = d_t * inv
        rs = jnp.concatenate([r_t, s_t], axis=0)
        ppre = lax.dot_general(m_ref[t], rs,
                               (((0,), (0,)), ((), ())),
                               preferred_element_type=jnp.float32)
        p = jnp.maximum(ppre + bo, 0.0)
        pred = pred + lax.dot_general(w3_ref[t], p,
                                      (((0,), (0,)), ((), ())),
                                      preferred_element_type=jnp.float32)
    out_ref[...] = pred + cst_ref[...]


def _tc_tail(o3, x2, m, w3, u, cst, bo):
    grid = (_N // _NB,)
    return pl.pallas_call(
        _tc_tail_body,
        out_shape=jax.ShapeDtypeStruct((_HOR, _N), jnp.float32),
        grid=grid,
        in_specs=[
            pl.BlockSpec((32, _T, _NB), lambda i: (0, 0, i)),
            pl.BlockSpec((_T, _NB), lambda i: (0, i)),
            pl.BlockSpec((_T, 16, _H), lambda i: (0, 0, 0)),
            pl.BlockSpec((_T, _H, _HOR), lambda i: (0, 0, 0)),
            pl.BlockSpec((_T, _HOR), lambda i: (0, 0)),
            pl.BlockSpec((_HOR, 1), lambda i: (0, 0)),
            pl.BlockSpec((_H, 1), lambda i: (0, 0)),
        ],
        out_specs=pl.BlockSpec((_HOR, _NB), lambda i: (0, i)),
    )(o3, x2, m, w3, u, cst, bo)


def _sinusoidal_pe(positions, d):
    pos = positions[:, None].astype(jnp.float32)
    i = jnp.arange(d // 2, dtype=jnp.float32)[None, :]
    angles = pos / jnp.power(10000.0, 2.0 * i / d)
    return jnp.concatenate([jnp.sin(angles), jnp.cos(angles)], axis=-1)


def kernel(x, edge_index, edge_weight, fq_param, W_enc, b_enc, W_gat,
           a_src, a_dst, W_out, b_out, W_dec, b_dec):
    x2 = x[0, :, :, 0]
    src = edge_index[0]
    dst = edge_index[1]

    pe = _sinusoidal_pe(jnp.arange(_T), _H)
    gv = W_enc[0] @ W_gat
    c = (b_enc[None, :] + pe) @ W_gat
    gh = gv.reshape(_NH, _DH)
    ch = c.reshape(_T, _NH, _DH)
    As = (gh * a_src).sum(-1)
    Ad = (gh * a_dst).sum(-1)
    Bq = (ch * a_src).sum(-1) + (ch * a_dst).sum(-1)
    Mx = x2.max(1)
    mx = x2.min(1)
    zmax = (jnp.where(As[None, :] > 0, As[None, :] * Mx[:, None], As[None, :] * mx[:, None])
            + jnp.where(Ad[None, :] > 0, Ad[None, :] * Mx[:, None], Ad[None, :] * mx[:, None])
            + Bq)
    mM = jnp.maximum(zmax, 0.2 * zmax)

    Wo3 = W_out.reshape(_NH, _DH, _H)
    G2 = jnp.einsum('hd,hdo->ho', gh, Wo3)
    C2 = jnp.einsum('thd,hdo->tho', ch, Wo3)
    W3 = W_dec.reshape(_T, _H, _HOR * _FIN)
    u = jnp.einsum('k,tko->to', W_enc[0], W3)
    cst = jnp.einsum('tk,tko->o', b_enc[None, :] + pe, W3) + b_dec

    stacked = jnp.stack([
        jnp.broadcast_to(As[None, :], (_T, _NH)), jnp.broadcast_to(Ad[None, :], (_T, _NH)),
        Bq, mM], axis=-1)
    par = jnp.broadcast_to(
        stacked.reshape(_T, 2, 4, 4)[..., None], (_T, 2, 4, 4, 16)
    ).reshape(_T * 2 * 16, 16)

    o = _sc_edge_pass(x2, src, dst, edge_weight, par)
    o3 = o.reshape(32, 8, _NP)

    m = jnp.concatenate([jnp.broadcast_to(G2[None], (_T, _NH, _H)), C2], axis=1)

    pred = _tc_tail(o3, x2, m, W3, u, cst.reshape(_HOR, 1), b_out.reshape(_H, 1))
    return pred.reshape(1, _HOR, _N, _FIN)


# speedup vs baseline: 1.0880x; 1.0086x over previous
import functools
import jax
import jax.numpy as jnp
import numpy as np
from jax import lax
from jax.experimental import pallas as pl
from jax.experimental.pallas import tpu as pltpu, tpu_sc as plsc

_B, _T, _N, _FIN = 1, 8, 10000, 1
_E = 160000
_H = 128
_NH = 8
_DH = _H // _NH
_HOR = 8

_NP = 10000
_C = 2000
_EHALF = _E // 2
_NCH = _EHALF // _C
_STEPS = _C // 16
_NB = _N
_NCORES = 2
_NSUB = 16


def _sc_edge_body(x_hbm, src_hbm, dst_hbm, ew_hbm, par_hbm, o_hbm,
                  x_v, src_v0, src_v1, dst_v0, dst_v1, ew_v0, ew_v1, par_v, tab_v,
                  sem_s0, sem_d0, sem_w0, sem_s1, sem_d1, sem_w1):
    cid = lax.axis_index("c")
    sid = lax.axis_index("s")
    wid = sid * _NCORES + cid
    hh = wid // 16
    rem = wid - hh * 16
    eh = rem // 8
    t = rem - eh * 8
    sems = ((sem_s0, sem_d0, sem_w0), (sem_s1, sem_d1, sem_w1))
    bufs = ((src_v0, dst_v0, ew_v0), (src_v1, dst_v1, ew_v1))

    e0 = eh * _EHALF

    def start(ci, b):
        off = e0 + ci * _C
        pltpu.async_copy(src_hbm.at[pl.ds(off, _C)], bufs[b][0], sems[b][0])
        pltpu.async_copy(dst_hbm.at[pl.ds(off, _C)], bufs[b][1], sems[b][1])
        pltpu.async_copy(ew_hbm.at[pl.ds(off, _C)], bufs[b][2], sems[b][2])

    def wait(b):
        pltpu.make_async_copy(src_hbm.at[pl.ds(0, _C)], bufs[b][0], sems[b][0]).wait()
        pltpu.make_async_copy(dst_hbm.at[pl.ds(0, _C)], bufs[b][1], sems[b][1]).wait()
        pltpu.make_async_copy(ew_hbm.at[pl.ds(0, _C)], bufs[b][2], sems[b][2]).wait()

    start(0, 0)
    start(1, 1)
    pltpu.sync_copy(x_hbm.at[pl.ds(t * _NP, _NP)], x_v)
    pltpu.sync_copy(par_hbm.at[pl.ds((t * 2 + hh) * 16, 16)], par_v)

    zeros = jnp.zeros((16,), jnp.float32)
    lanes = lax.iota(jnp.int32, 16)

    @plsc.parallel_loop(0, _NP // 16, unroll=8)
    def _zero(i):
        for k in range(8):
            tab_v[k, pl.ds(i * 16, 16)] = zeros

    def pair_body(pi, carry):
        for b in range(2):
            ci = pi * 2 + b
            wait(b)

            @plsc.parallel_loop(0, _STEPS, unroll=4)
            def _step(s):
                sv = bufs[b][0][pl.ds(s * 16, 16)]
                dv = bufs[b][1][pl.ds(s * 16, 16)]
                ewv = bufs[b][2][pl.ds(s * 16, 16)]
                xs = plsc.load_gather(x_v, [sv])
                xd = plsc.load_gather(x_v, [dv])
                for j in range(4):
                    asv = par_v[j * 4 + 0]
                    adv = par_v[j * 4 + 1]
                    bqv = par_v[j * 4 + 2]
                    mmv = par_v[j * 4 + 3]
                    z = xs * asv + xd * adv + bqv
                    zl = jnp.maximum(z, 0.2 * z)
                    w = jnp.exp(zl - mmv) * ewv
                    plsc.addupdate_scatter(tab_v, [jnp.full((16,), j, jnp.int32), dv], w)
                    plsc.addupdate_scatter(tab_v, [jnp.full((16,), 4 + j, jnp.int32), dv], w * xs)

            @pl.when(ci + 2 < _NCH)
            def _():
                start(ci + 2, b)
        return carry

    lax.fori_loop(0, _NCH // 2, pair_body, 0)
    pltpu.sync_copy(tab_v, o_hbm.at[wid])


def _sc_edge_pass(x2, src, dst, ew, par):
    mesh = plsc.VectorSubcoreMesh(core_axis_name="c", subcore_axis_name="s")
    f = pl.kernel(
        _sc_edge_body,
        out_type=jax.ShapeDtypeStruct((32, 8, _NP), jnp.float32),
        mesh=mesh,
        scratch_types=[
            pltpu.VMEM((_NP,), jnp.float32),
            pltpu.VMEM((_C,), jnp.int32),
            pltpu.VMEM((_C,), jnp.int32),
            pltpu.VMEM((_C,), jnp.int32),
            pltpu.VMEM((_C,), jnp.int32),
            pltpu.VMEM((_C,), jnp.float32),
            pltpu.VMEM((_C,), jnp.float32),
            pltpu.VMEM((16, 16), jnp.float32),
            pltpu.VMEM((8, _NP), jnp.float32),
            pltpu.SemaphoreType.DMA,
            pltpu.SemaphoreType.DMA,
            pltpu.SemaphoreType.DMA,
            pltpu.SemaphoreType.DMA,
            pltpu.SemaphoreType.DMA,
            pltpu.SemaphoreType.DMA,
        ],
        compiler_params=pltpu.CompilerParams(needs_layout_passes=False),
    )
    return f(x2.reshape(-1), src, dst, ew, par)


def _tc_tail_body(o_ref, x_ref, m_ref, w3_ref, u_ref, cst_ref, bo_ref, out_ref):
    ob = o_ref[...]
    o5 = ob.reshape(2, 2, _T, 8, _NB)
    acc = o5[:, 0] + o5[:, 1]
    den = acc[:, :, 0:4, :]
    num = acc[:, :, 4:8, :]
    bo = bo_ref[...]
    pred = lax.dot_general(u_ref[...], x_ref[...],
                           (((0,), (0,)), ((), ())),
                           preferred_element_type=jnp.float32)
    for t in range(_T):
        d_t = den[:, t].reshape(_NH, _NB)
        n_t = num[:, t].reshape(_NH, _NB)
        inv = 1.0 / (d_t + 1e-16)
        r_t = n_t * inv
        s_t = d_t * inv
        rs = jnp.concatenate([r_t, s_t], axis=0)
        ppre = lax.dot_general(m_ref[t], rs,
                               (((0,), (0,)), ((), ())),
                               preferred_element_type=jnp.float32)
        p = jnp.maximum(ppre + bo, 0.0)
        pred = pred + lax.dot_general(w3_ref[t], p,
                                      (((0,), (0,)), ((), ())),
                                      preferred_element_type=jnp.float32)
    out_ref[...] = pred + cst_ref[...]


def _tc_tail(o3, x2, m, w3, u, cst, bo):
    grid = (_N // _NB,)
    return pl.pallas_call(
        _tc_tail_body,
        out_shape=jax.ShapeDtypeStruct((_HOR, _N), jnp.float32),
        grid=grid,
        in_specs=[
            pl.BlockSpec((32, _T, _NB), lambda i: (0, 0, i)),
            pl.BlockSpec((_T, _NB), lambda i: (0, i)),
            pl.BlockSpec((_T, 16, _H), lambda i: (0, 0, 0)),
            pl.BlockSpec((_T, _H, _HOR), lambda i: (0, 0, 0)),
            pl.BlockSpec((_T, _HOR), lambda i: (0, 0)),
            pl.BlockSpec((_HOR, 1), lambda i: (0, 0)),
            pl.BlockSpec((_H, 1), lambda i: (0, 0)),
        ],
        out_specs=pl.BlockSpec((_HOR, _NB), lambda i: (0, i)),
    )(o3, x2, m, w3, u, cst, bo)


def _sinusoidal_pe(positions, d):
    pos = positions[:, None].astype(jnp.float32)
    i = jnp.arange(d // 2, dtype=jnp.float32)[None, :]
    angles = pos / jnp.power(10000.0, 2.0 * i / d)
    return jnp.concatenate([jnp.sin(angles), jnp.cos(angles)], axis=-1)


def kernel(x, edge_index, edge_weight, fq_param, W_enc, b_enc, W_gat,
           a_src, a_dst, W_out, b_out, W_dec, b_dec):
    x2 = x[0, :, :, 0]
    src = edge_index[0]
    dst = edge_index[1]

    pe = _sinusoidal_pe(jnp.arange(_T), _H)
    gv = W_enc[0] @ W_gat
    c = (b_enc[None, :] + pe) @ W_gat
    gh = gv.reshape(_NH, _DH)
    ch = c.reshape(_T, _NH, _DH)
    As = (gh * a_src).sum(-1)
    Ad = (gh * a_dst).sum(-1)
    Bq = (ch * a_src).sum(-1) + (ch * a_dst).sum(-1)
    Mx = x2.max(1)
    mx = x2.min(1)
    zmax = (jnp.where(As[None, :] > 0, As[None, :] * Mx[:, None], As[None, :] * mx[:, None])
            + jnp.where(Ad[None, :] > 0, Ad[None, :] * Mx[:, None], Ad[None, :] * mx[:, None])
            + Bq)
    mM = jnp.maximum(zmax, 0.2 * zmax)

    Wo3 = W_out.reshape(_NH, _DH, _H)
    G2 = jnp.einsum('hd,hdo->ho', gh, Wo3)
    C2 = jnp.einsum('thd,hdo->tho', ch, Wo3)
    W3 = W_dec.reshape(_T, _H, _HOR * _FIN)
    u = jnp.einsum('k,tko->to', W_enc[0], W3)
    cst = jnp.einsum('tk,tko->o', b_enc[None, :] + pe, W3) + b_dec

    stacked = jnp.stack([
        jnp.broadcast_to(As[None, :], (_T, _NH)), jnp.broadcast_to(Ad[None, :], (_T, _NH)),
        Bq, mM], axis=-1)
    par = jnp.broadcast_to(
        stacked.reshape(_T, 2, 4, 4)[..., None], (_T, 2, 4, 4, 16)
    ).reshape(_T * 2 * 16, 16)

    o3 = _sc_edge_pass(x2, src, dst, edge_weight, par)

    m = jnp.concatenate([jnp.broadcast_to(G2[None], (_T, _NH, _H)), C2], axis=1)

    pred = _tc_tail(o3, x2, m, W3, u, cst.reshape(_HOR, 1), b_out.reshape(_H, 1))
    return pred.reshape(1, _HOR, _N, _FIN)


# speedup vs baseline: 1.1069x; 1.0174x over previous
import functools
import jax
import jax.numpy as jnp
import numpy as np
from jax import lax
from jax.experimental import pallas as pl
from jax.experimental.pallas import tpu as pltpu, tpu_sc as plsc

_B, _T, _N, _FIN = 1, 8, 10000, 1
_E = 160000
_H = 128
_NH = 8
_DH = _H // _NH
_HOR = 8

_NP = 10000
_NT = 10240
_C = 2000
_EHALF = _E // 2
_NCH = _EHALF // _C
_STEPS = _C // 16
_NB = 1024
_NCORES = 2
_NSUB = 16


def _sc_edge_body(x_hbm, src_hbm, dst_hbm, ew_hbm, par_hbm,
                  o0, o1, o2, o3, o4, o5, o6, o7,
                  x_v, src_v0, src_v1, dst_v0, dst_v1, ew_v0, ew_v1, par_v,
                  tb0, tb1, tb2, tb3, tb4, tb5, tb6, tb7,
                  sem_s0, sem_d0, sem_w0, sem_s1, sem_d1, sem_w1):
    tabs = (tb0, tb1, tb2, tb3, tb4, tb5, tb6, tb7)
    cid = lax.axis_index("c")
    sid = lax.axis_index("s")
    wid = sid * _NCORES + cid
    hh = wid // 16
    rem = wid - hh * 16
    eh = rem // 8
    t = rem - eh * 8
    sems = ((sem_s0, sem_d0, sem_w0), (sem_s1, sem_d1, sem_w1))
    bufs = ((src_v0, dst_v0, ew_v0), (src_v1, dst_v1, ew_v1))

    e0 = eh * _EHALF

    def start(ci, b):
        off = e0 + ci * _C
        pltpu.async_copy(src_hbm.at[pl.ds(off, _C)], bufs[b][0], sems[b][0])
        pltpu.async_copy(dst_hbm.at[pl.ds(off, _C)], bufs[b][1], sems[b][1])
        pltpu.async_copy(ew_hbm.at[pl.ds(off, _C)], bufs[b][2], sems[b][2])

    def wait(b):
        pltpu.make_async_copy(src_hbm.at[pl.ds(0, _C)], bufs[b][0], sems[b][0]).wait()
        pltpu.make_async_copy(dst_hbm.at[pl.ds(0, _C)], bufs[b][1], sems[b][1]).wait()
        pltpu.make_async_copy(ew_hbm.at[pl.ds(0, _C)], bufs[b][2], sems[b][2]).wait()

    start(0, 0)
    start(1, 1)
    pltpu.sync_copy(x_hbm.at[pl.ds(t * _NP, _NP)], x_v)
    pltpu.sync_copy(par_hbm.at[pl.ds((t * 2 + hh) * 16, 16)], par_v)

    zeros = jnp.zeros((16,), jnp.float32)
    lanes = lax.iota(jnp.int32, 16)

    @plsc.parallel_loop(0, _NT // 16, unroll=8)
    def _zero(i):
        for k in range(8):
            tabs[k][pl.ds(i * 16, 16)] = zeros

    def pair_body(pi, carry):
        for b in range(2):
            ci = pi * 2 + b
            wait(b)

            @plsc.parallel_loop(0, _STEPS, unroll=4)
            def _step(s):
                sv = bufs[b][0][pl.ds(s * 16, 16)]
                dv = bufs[b][1][pl.ds(s * 16, 16)]
                ewv = bufs[b][2][pl.ds(s * 16, 16)]
                xs = plsc.load_gather(x_v, [sv])
                xd = plsc.load_gather(x_v, [dv])
                for j in range(4):
                    asv = par_v[j * 4 + 0]
                    adv = par_v[j * 4 + 1]
                    bqv = par_v[j * 4 + 2]
                    mmv = par_v[j * 4 + 3]
                    z = xs * asv + xd * adv + bqv
                    zl = jnp.maximum(z, 0.2 * z)
                    w = jnp.exp(zl - mmv) * ewv
                    plsc.addupdate_scatter(tabs[j], [dv], w)
                    plsc.addupdate_scatter(tabs[4 + j], [dv], w * xs)

            @pl.when(ci + 2 < _NCH)
            def _():
                start(ci + 2, b)
        return carry

    lax.fori_loop(0, _NCH // 2, pair_body, 0)
    for k, o_k in enumerate((o0, o1, o2, o3, o4, o5, o6, o7)):
        pltpu.sync_copy(tabs[k], o_k.at[wid])


def _sc_edge_pass(x2, src, dst, ew, par):
    mesh = plsc.VectorSubcoreMesh(core_axis_name="c", subcore_axis_name="s")
    f = pl.kernel(
        _sc_edge_body,
        out_type=[jax.ShapeDtypeStruct((32, _NT), jnp.float32)] * 8,
        mesh=mesh,
        scratch_types=[
            pltpu.VMEM((_NP,), jnp.float32),
            pltpu.VMEM((_C,), jnp.int32),
            pltpu.VMEM((_C,), jnp.int32),
            pltpu.VMEM((_C,), jnp.int32),
            pltpu.VMEM((_C,), jnp.int32),
            pltpu.VMEM((_C,), jnp.float32),
            pltpu.VMEM((_C,), jnp.float32),
            pltpu.VMEM((16, 16), jnp.float32),
            pltpu.VMEM((_NT,), jnp.float32),
            pltpu.VMEM((_NT,), jnp.float32),
            pltpu.VMEM((_NT,), jnp.float32),
            pltpu.VMEM((_NT,), jnp.float32),
            pltpu.VMEM((_NT,), jnp.float32),
            pltpu.VMEM((_NT,), jnp.float32),
            pltpu.VMEM((_NT,), jnp.float32),
            pltpu.VMEM((_NT,), jnp.float32),
            pltpu.SemaphoreType.DMA,
            pltpu.SemaphoreType.DMA,
            pltpu.SemaphoreType.DMA,
            pltpu.SemaphoreType.DMA,
            pltpu.SemaphoreType.DMA,
            pltpu.SemaphoreType.DMA,
        ],
        compiler_params=pltpu.CompilerParams(needs_layout_passes=False),
    )
    return f(x2.reshape(-1), src, dst, ew, par)


def _tc_tail_body(o0_ref, o1_ref, o2_ref, o3_ref, o4_ref, o5_ref, o6_ref, o7_ref,
                  x_ref, m_ref, w3_ref, u_ref, cst_ref, bo_ref, out_ref):
    o_refs = (o0_ref, o1_ref, o2_ref, o3_ref, o4_ref, o5_ref, o6_ref, o7_ref)
    accs = []
    for k in range(8):
        ok = o_refs[k][...].reshape(2, 2, _T, _NB)
        accs.append(ok[:, 0] + ok[:, 1])
    bo = bo_ref[...]
    pred = lax.dot_general(u_ref[...], x_ref[...],
                           (((0,), (0,)), ((), ())),
                           preferred_element_type=jnp.float32)
    for t in range(_T):
        d_t = jnp.stack([accs[h % 4][h // 4, t] for h in range(_NH)], axis=0)
        n_t = jnp.stack([accs[4 + h % 4][h // 4, t] for h in range(_NH)], axis=0)
        inv = 1.0 / (d_t + 1e-16)
        r_t = n_t * inv
        s_t = d_t * inv
        rs = jnp.concatenate([r_t, s_t], axis=0)
        ppre = lax.dot_general(m_ref[t], rs,
                               (((0,), (0,)), ((), ())),
                               preferred_element_type=jnp.float32)
        p = jnp.maximum(ppre + bo, 0.0)
        pred = pred + lax.dot_general(w3_ref[t], p,
                                      (((0,), (0,)), ((), ())),
                                      preferred_element_type=jnp.float32)
    out_ref[...] = pred + cst_ref[...]


def _tc_tail(olist, x2, m, w3, u, cst, bo):
    grid = (_NT // _NB,)
    return pl.pallas_call(
        _tc_tail_body,
        out_shape=jax.ShapeDtypeStruct((_HOR, _N), jnp.float32),
        grid=grid,
        in_specs=[pl.BlockSpec((32, _NB), lambda i: (0, i)) for _ in range(8)] + [
            pl.BlockSpec((_T, _NB), lambda i: (0, i)),
            pl.BlockSpec((_T, 16, _H), lambda i: (0, 0, 0)),
            pl.BlockSpec((_T, _H, _HOR), lambda i: (0, 0, 0)),
            pl.BlockSpec((_T, _HOR), lambda i: (0, 0)),
            pl.BlockSpec((_HOR, 1), lambda i: (0, 0)),
            pl.BlockSpec((_H, 1), lambda i: (0, 0)),
        ],
        out_specs=pl.BlockSpec((_HOR, _NB), lambda i: (0, i)),
    )(*olist, x2, m, w3, u, cst, bo)


def _sinusoidal_pe(positions, d):
    pos = positions[:, None].astype(jnp.float32)
    i = jnp.arange(d // 2, dtype=jnp.float32)[None, :]
    angles = pos / jnp.power(10000.0, 2.0 * i / d)
    return jnp.concatenate([jnp.sin(angles), jnp.cos(angles)], axis=-1)


def kernel(x, edge_index, edge_weight, fq_param, W_enc, b_enc, W_gat,
           a_src, a_dst, W_out, b_out, W_dec, b_dec):
    x2 = x[0, :, :, 0]
    src = edge_index[0]
    dst = edge_index[1]

    pe = _sinusoidal_pe(jnp.arange(_T), _H)
    gv = W_enc[0] @ W_gat
    c = (b_enc[None, :] + pe) @ W_gat
    gh = gv.reshape(_NH, _DH)
    ch = c.reshape(_T, _NH, _DH)
    As = (gh * a_src).sum(-1)
    Ad = (gh * a_dst).sum(-1)
    Bq = (ch * a_src).sum(-1) + (ch * a_dst).sum(-1)
    Mx = x2.max(1)
    mx = x2.min(1)
    zmax = (jnp.where(As[None, :] > 0, As[None, :] * Mx[:, None], As[None, :] * mx[:, None])
            + jnp.where(Ad[None, :] > 0, Ad[None, :] * Mx[:, None], Ad[None, :] * mx[:, None])
            + Bq)
    mM = jnp.maximum(zmax, 0.2 * zmax)

    Wo3 = W_out.reshape(_NH, _DH, _H)
    G2 = jnp.einsum('hd,hdo->ho', gh, Wo3)
    C2 = jnp.einsum('thd,hdo->tho', ch, Wo3)
    W3 = W_dec.reshape(_T, _H, _HOR * _FIN)
    u = jnp.einsum('k,tko->to', W_enc[0], W3)
    cst = jnp.einsum('tk,tko->o', b_enc[None, :] + pe, W3) + b_dec

    stacked = jnp.stack([
        jnp.broadcast_to(As[None, :], (_T, _NH)), jnp.broadcast_to(Ad[None, :], (_T, _NH)),
        Bq, mM], axis=-1)
    par = jnp.broadcast_to(
        stacked.reshape(_T, 2, 4, 4)[..., None], (_T, 2, 4, 4, 16)
    ).reshape(_T * 2 * 16, 16)

    olist = _sc_edge_pass(x2, src, dst, edge_weight, par)

    m = jnp.concatenate([jnp.broadcast_to(G2[None], (_T, _NH, _H)), C2], axis=1)

    pred = _tc_tail(olist, x2, m, W3, u, cst.reshape(_HOR, 1), b_out.reshape(_H, 1))
    return pred.reshape(1, _HOR, _N, _FIN)


# speedup vs baseline: 1.1524x; 1.0411x over previous
import functools
import jax
import jax.numpy as jnp
import numpy as np
from jax import lax
from jax.experimental import pallas as pl
from jax.experimental.pallas import tpu as pltpu, tpu_sc as plsc

_B, _T, _N, _FIN = 1, 8, 10000, 1
_E = 160000
_H = 128
_NH = 8
_DH = _H // _NH
_HOR = 8

_NP = 10000
_NT = 10240
_C = 2000
_EHALF = _E // 2
_NCH = _EHALF // _C
_STEPS = _C // 16
_NB = 1024
_NCORES = 2
_NSUB = 16


def _sc_edge_body(x_hbm, src_hbm, dst_hbm, ew_hbm, par_hbm,
                  o0, o1, o2, o3, o4, o5, o6, o7,
                  x_v, src_v0, src_v1, dst_v0, dst_v1, ew_v0, ew_v1, par_v,
                  tb0, tb1, tb2, tb3, tb4, tb5, tb6, tb7,
                  sem_s0, sem_d0, sem_w0, sem_s1, sem_d1, sem_w1):
    tabs = (tb0, tb1, tb2, tb3, tb4, tb5, tb6, tb7)
    cid = lax.axis_index("c")
    sid = lax.axis_index("s")
    wid = sid * _NCORES + cid
    hh = wid // 16
    rem = wid - hh * 16
    eh = rem // 8
    t = rem - eh * 8
    sems = ((sem_s0, sem_d0, sem_w0), (sem_s1, sem_d1, sem_w1))
    bufs = ((src_v0, dst_v0, ew_v0), (src_v1, dst_v1, ew_v1))

    e0 = eh * _EHALF

    def start(ci, b):
        off = e0 + ci * _C
        pltpu.async_copy(src_hbm.at[pl.ds(off, _C)], bufs[b][0], sems[b][0])
        pltpu.async_copy(dst_hbm.at[pl.ds(off, _C)], bufs[b][1], sems[b][1])
        pltpu.async_copy(ew_hbm.at[pl.ds(off, _C)], bufs[b][2], sems[b][2])

    def wait(b):
        pltpu.make_async_copy(src_hbm.at[pl.ds(0, _C)], bufs[b][0], sems[b][0]).wait()
        pltpu.make_async_copy(dst_hbm.at[pl.ds(0, _C)], bufs[b][1], sems[b][1]).wait()
        pltpu.make_async_copy(ew_hbm.at[pl.ds(0, _C)], bufs[b][2], sems[b][2]).wait()

    start(0, 0)
    start(1, 1)
    pltpu.sync_copy(x_hbm.at[pl.ds(t * _NP, _NP)], x_v)
    pltpu.sync_copy(par_hbm.at[pl.ds((t * 2 + hh) * 16, 16)], par_v)

    zeros = jnp.zeros((16,), jnp.float32)
    lanes = lax.iota(jnp.int32, 16)

    @plsc.parallel_loop(0, _NT // 16, unroll=8)
    def _zero(i):
        for k in range(8):
            tabs[k][pl.ds(i * 16, 16)] = zeros

    def pair_body(pi, carry):
        for b in range(2):
            ci = pi * 2 + b
            wait(b)

            @plsc.parallel_loop(0, _STEPS, unroll=4)
            def _step(s):
                sv = bufs[b][0][pl.ds(s * 16, 16)]
                dv = bufs[b][1][pl.ds(s * 16, 16)]
                ewv = bufs[b][2][pl.ds(s * 16, 16)]
                xs = plsc.load_gather(x_v, [sv])
                xd = plsc.load_gather(x_v, [dv])
                for j in range(4):
                    asv = par_v[j * 4 + 0]
                    adv = par_v[j * 4 + 1]
                    bqv = par_v[j * 4 + 2]
                    mmv = par_v[j * 4 + 3]
                    z = xs * asv + xd * adv + bqv
                    zl = jnp.maximum(z, 0.2 * z)
                    w = jnp.exp(zl - mmv) * ewv
                    plsc.addupdate_scatter(tabs[j], [dv], w)
                    plsc.addupdate_scatter(tabs[4 + j], [dv], w * xs)

            @pl.when(ci + 2 < _NCH)
            def _():
                start(ci + 2, b)
        return carry

    lax.fori_loop(0, _NCH // 2, pair_body, 0)
    for k, o_k in enumerate((o0, o1, o2, o3, o4, o5, o6, o7)):
        pltpu.sync_copy(tabs[k], o_k.at[wid])


def _sc_edge_pass(x2, src, dst, ew, par):
    mesh = plsc.VectorSubcoreMesh(core_axis_name="c", subcore_axis_name="s")
    f = pl.kernel(
        _sc_edge_body,
        out_type=[jax.ShapeDtypeStruct((32, _NT), jnp.float32)] * 8,
        mesh=mesh,
        scratch_types=[
            pltpu.VMEM((_NP,), jnp.float32),
            pltpu.VMEM((_C,), jnp.int32),
            pltpu.VMEM((_C,), jnp.int32),
            pltpu.VMEM((_C,), jnp.int32),
            pltpu.VMEM((_C,), jnp.int32),
            pltpu.VMEM((_C,), jnp.float32),
            pltpu.VMEM((_C,), jnp.float32),
            pltpu.VMEM((16, 16), jnp.float32),
            pltpu.VMEM((_NT,), jnp.float32),
            pltpu.VMEM((_NT,), jnp.float32),
            pltpu.VMEM((_NT,), jnp.float32),
            pltpu.VMEM((_NT,), jnp.float32),
            pltpu.VMEM((_NT,), jnp.float32),
            pltpu.VMEM((_NT,), jnp.float32),
            pltpu.VMEM((_NT,), jnp.float32),
            pltpu.VMEM((_NT,), jnp.float32),
            pltpu.SemaphoreType.DMA,
            pltpu.SemaphoreType.DMA,
            pltpu.SemaphoreType.DMA,
            pltpu.SemaphoreType.DMA,
            pltpu.SemaphoreType.DMA,
            pltpu.SemaphoreType.DMA,
        ],
        compiler_params=pltpu.CompilerParams(needs_layout_passes=False),
    )
    return f(x2.reshape(-1), src, dst, ew, par)


def _tc_tail_body(o0_ref, o1_ref, o2_ref, o3_ref, o4_ref, o5_ref, o6_ref, o7_ref,
                  x_ref, m_ref, w3_ref, u_ref, cst_ref, bo_ref, out_ref):
    o_refs = (o0_ref, o1_ref, o2_ref, o3_ref, o4_ref, o5_ref, o6_ref, o7_ref)
    accs = []
    for k in range(8):
        ok = o_refs[k][...].reshape(2, 2, _T, _NB)
        accs.append(ok[:, 0] + ok[:, 1])
    bo = bo_ref[...]
    pred = lax.dot_general(u_ref[...], x_ref[...],
                           (((0,), (0,)), ((), ())),
                           preferred_element_type=jnp.float32)
    d_all = jnp.concatenate([accs[j].transpose(1, 0, 2) for j in range(4)],
                            axis=1)
    n_all = jnp.concatenate([accs[4 + j].transpose(1, 0, 2) for j in range(4)],
                            axis=1)
    inv = 1.0 / (d_all + 1e-16)
    rs = jnp.concatenate([n_all * inv, d_all * inv], axis=1)
    ppre = lax.dot_general(m_ref[...], rs,
                           (((1,), (1,)), ((0,), (0,))),
                           preferred_element_type=jnp.float32)
    p = jnp.maximum(ppre + bo[None], 0.0)
    pw = lax.dot_general(w3_ref[...], p,
                         (((1,), (1,)), ((0,), (0,))),
                         preferred_element_type=jnp.float32)
    out_ref[...] = pred + pw.sum(axis=0) + cst_ref[...]


def _tc_tail(olist, x2, m, w3, u, cst, bo):
    grid = (_NT // _NB,)
    return pl.pallas_call(
        _tc_tail_body,
        out_shape=jax.ShapeDtypeStruct((_HOR, _N), jnp.float32),
        grid=grid,
        in_specs=[pl.BlockSpec((32, _NB), lambda i: (0, i)) for _ in range(8)] + [
            pl.BlockSpec((_T, _NB), lambda i: (0, i)),
            pl.BlockSpec((_T, 16, _H), lambda i: (0, 0, 0)),
            pl.BlockSpec((_T, _H, _HOR), lambda i: (0, 0, 0)),
            pl.BlockSpec((_T, _HOR), lambda i: (0, 0)),
            pl.BlockSpec((_HOR, 1), lambda i: (0, 0)),
            pl.BlockSpec((_H, 1), lambda i: (0, 0)),
        ],
        out_specs=pl.BlockSpec((_HOR, _NB), lambda i: (0, i)),
    )(*olist, x2, m, w3, u, cst, bo)


def _sinusoidal_pe(positions, d):
    pos = positions[:, None].astype(jnp.float32)
    i = jnp.arange(d // 2, dtype=jnp.float32)[None, :]
    angles = pos / jnp.power(10000.0, 2.0 * i / d)
    return jnp.concatenate([jnp.sin(angles), jnp.cos(angles)], axis=-1)


def kernel(x, edge_index, edge_weight, fq_param, W_enc, b_enc, W_gat,
           a_src, a_dst, W_out, b_out, W_dec, b_dec):
    x2 = x[0, :, :, 0]
    src = edge_index[0]
    dst = edge_index[1]

    pe = _sinusoidal_pe(jnp.arange(_T), _H)
    gv = W_enc[0] @ W_gat
    c = (b_enc[None, :] + pe) @ W_gat
    gh = gv.reshape(_NH, _DH)
    ch = c.reshape(_T, _NH, _DH)
    As = (gh * a_src).sum(-1)
    Ad = (gh * a_dst).sum(-1)
    Bq = (ch * a_src).sum(-1) + (ch * a_dst).sum(-1)
    Mx = x2.max(1)
    mx = x2.min(1)
    zmax = (jnp.where(As[None, :] > 0, As[None, :] * Mx[:, None], As[None, :] * mx[:, None])
            + jnp.where(Ad[None, :] > 0, Ad[None, :] * Mx[:, None], Ad[None, :] * mx[:, None])
            + Bq)
    mM = jnp.maximum(zmax, 0.2 * zmax)

    Wo3 = W_out.reshape(_NH, _DH, _H)
    G2 = jnp.einsum('hd,hdo->ho', gh, Wo3)
    C2 = jnp.einsum('thd,hdo->tho', ch, Wo3)
    horder = jnp.array([0, 4, 1, 5, 2, 6, 3, 7])
    G2 = G2[horder]
    C2 = C2[:, horder]
    W3 = W_dec.reshape(_T, _H, _HOR * _FIN)
    u = jnp.einsum('k,tko->to', W_enc[0], W3)
    cst = jnp.einsum('tk,tko->o', b_enc[None, :] + pe, W3) + b_dec

    stacked = jnp.stack([
        jnp.broadcast_to(As[None, :], (_T, _NH)), jnp.broadcast_to(Ad[None, :], (_T, _NH)),
        Bq, mM], axis=-1)
    par = jnp.broadcast_to(
        stacked.reshape(_T, 2, 4, 4)[..., None], (_T, 2, 4, 4, 16)
    ).reshape(_T * 2 * 16, 16)

    olist = _sc_edge_pass(x2, src, dst, edge_weight, par)

    m = jnp.concatenate([jnp.broadcast_to(G2[None], (_T, _NH, _H)), C2], axis=1)

    pred = _tc_tail(olist, x2, m, W3, u, cst.reshape(_HOR, 1), b_out.reshape(_H, 1))
    return pred.reshape(1, _HOR, _N, _FIN)


# speedup vs baseline: 1.1768x; 1.0212x over previous
import functools
import jax
import jax.numpy as jnp
import numpy as np
from jax import lax
from jax.experimental import pallas as pl
from jax.experimental.pallas import tpu as pltpu, tpu_sc as plsc

_B, _T, _N, _FIN = 1, 8, 10000, 1
_E = 160000
_H = 128
_NH = 8
_DH = _H // _NH
_HOR = 8

_NP = 10000
_NT = 10240
_C = 2000
_EHALF = _E // 2
_NCH = _EHALF // _C
_STEPS = _C // 16
_NB = 1024
_NCORES = 2
_NSUB = 16


def _sc_edge_body(x_hbm, src_hbm, dst_hbm, ew_hbm, par_hbm,
                  o0, o1, o2, o3, o4, o5, o6, o7,
                  x_v, src_v0, src_v1, dst_v0, dst_v1, ew_v0, ew_v1, par_v,
                  tb0, tb1, tb2, tb3, tb4, tb5, tb6, tb7,
                  sem_s0, sem_d0, sem_w0, sem_s1, sem_d1, sem_w1, sem_x, sem_p):
    tabs = (tb0, tb1, tb2, tb3, tb4, tb5, tb6, tb7)
    cid = lax.axis_index("c")
    sid = lax.axis_index("s")
    wid = sid * _NCORES + cid
    hh = wid // 16
    rem = wid - hh * 16
    eh = rem // 8
    t = rem - eh * 8
    sems = ((sem_s0, sem_d0, sem_w0), (sem_s1, sem_d1, sem_w1))
    bufs = ((src_v0, dst_v0, ew_v0), (src_v1, dst_v1, ew_v1))

    e0 = eh * _EHALF

    def start(ci, b):
        off = e0 + ci * _C
        pltpu.async_copy(src_hbm.at[pl.ds(off, _C)], bufs[b][0], sems[b][0])
        pltpu.async_copy(dst_hbm.at[pl.ds(off, _C)], bufs[b][1], sems[b][1])
        pltpu.async_copy(ew_hbm.at[pl.ds(off, _C)], bufs[b][2], sems[b][2])

    def wait(b):
        pltpu.make_async_copy(src_hbm.at[pl.ds(0, _C)], bufs[b][0], sems[b][0]).wait()
        pltpu.make_async_copy(dst_hbm.at[pl.ds(0, _C)], bufs[b][1], sems[b][1]).wait()
        pltpu.make_async_copy(ew_hbm.at[pl.ds(0, _C)], bufs[b][2], sems[b][2]).wait()

    start(0, 0)
    start(1, 1)
    cx = pltpu.async_copy(x_hbm.at[pl.ds(t * _NP, _NP)], x_v, sem_x)
    cp = pltpu.async_copy(par_hbm.at[pl.ds((t * 2 + hh) * 16, 16)], par_v, sem_p)

    zeros = jnp.zeros((16,), jnp.float32)
    lanes = lax.iota(jnp.int32, 16)

    @plsc.parallel_loop(0, _NT // 16, unroll=8)
    def _zero(i):
        for k in range(8):
            tabs[k][pl.ds(i * 16, 16)] = zeros

    cx.wait()
    cp.wait()

    def pair_body(pi, carry):
        for b in range(2):
            ci = pi * 2 + b
            wait(b)

            @plsc.parallel_loop(0, _STEPS, unroll=4)
            def _step(s):
                sv = bufs[b][0][pl.ds(s * 16, 16)]
                dv = bufs[b][1][pl.ds(s * 16, 16)]
                ewv = bufs[b][2][pl.ds(s * 16, 16)]
                xs = plsc.load_gather(x_v, [sv])
                xd = plsc.load_gather(x_v, [dv])
                for j in range(4):
                    asv = par_v[j * 4 + 0]
                    adv = par_v[j * 4 + 1]
                    bqv = par_v[j * 4 + 2]
                    mmv = par_v[j * 4 + 3]
                    z = xs * asv + xd * adv + bqv
                    zl = jnp.maximum(z, 0.2 * z)
                    w = jnp.exp(zl - mmv) * ewv
                    plsc.addupdate_scatter(tabs[j], [dv], w)
                    plsc.addupdate_scatter(tabs[4 + j], [dv], w * xs)

            @pl.when(ci + 2 < _NCH)
            def _():
                start(ci + 2, b)
        return carry

    lax.fori_loop(0, _NCH // 2, pair_body, 0)
    osems = (sem_s0, sem_d0, sem_w0, sem_s1, sem_d1, sem_w1, sem_x, sem_p)
    outs = (o0, o1, o2, o3, o4, o5, o6, o7)
    cps = [pltpu.async_copy(tabs[k], outs[k].at[wid], osems[k]) for k in range(8)]
    for c in cps:
        c.wait()


def _sc_edge_pass(x2, src, dst, ew, par):
    mesh = plsc.VectorSubcoreMesh(core_axis_name="c", subcore_axis_name="s")
    f = pl.kernel(
        _sc_edge_body,
        out_type=[jax.ShapeDtypeStruct((32, _NT), jnp.float32)] * 8,
        mesh=mesh,
        scratch_types=[
            pltpu.VMEM((_NP,), jnp.float32),
            pltpu.VMEM((_C,), jnp.int32),
            pltpu.VMEM((_C,), jnp.int32),
            pltpu.VMEM((_C,), jnp.int32),
            pltpu.VMEM((_C,), jnp.int32),
            pltpu.VMEM((_C,), jnp.float32),
            pltpu.VMEM((_C,), jnp.float32),
            pltpu.VMEM((16, 16), jnp.float32),
            pltpu.VMEM((_NT,), jnp.float32),
            pltpu.VMEM((_NT,), jnp.float32),
            pltpu.VMEM((_NT,), jnp.float32),
            pltpu.VMEM((_NT,), jnp.float32),
            pltpu.VMEM((_NT,), jnp.float32),
            pltpu.VMEM((_NT,), jnp.float32),
            pltpu.VMEM((_NT,), jnp.float32),
            pltpu.VMEM((_NT,), jnp.float32),
            pltpu.SemaphoreType.DMA,
            pltpu.SemaphoreType.DMA,
            pltpu.SemaphoreType.DMA,
            pltpu.SemaphoreType.DMA,
            pltpu.SemaphoreType.DMA,
            pltpu.SemaphoreType.DMA,
            pltpu.SemaphoreType.DMA,
            pltpu.SemaphoreType.DMA,
        ],
        compiler_params=pltpu.CompilerParams(needs_layout_passes=False),
    )
    return f(x2.reshape(-1), src, dst, ew, par)


def _tc_tail_body(o0_ref, o1_ref, o2_ref, o3_ref, o4_ref, o5_ref, o6_ref, o7_ref,
                  x_ref, m_ref, w3_ref, u_ref, cst_ref, bo_ref, out_ref):
    o_refs = (o0_ref, o1_ref, o2_ref, o3_ref, o4_ref, o5_ref, o6_ref, o7_ref)
    accs = []
    for k in range(8):
        ok = o_refs[k][...].reshape(2, 2, _T, _NB)
        accs.append(ok[:, 0] + ok[:, 1])
    bo = bo_ref[...]
    pred = lax.dot_general(u_ref[...], x_ref[...],
                           (((0,), (0,)), ((), ())),
                           preferred_element_type=jnp.float32)
    d_all = jnp.concatenate([accs[j].transpose(1, 0, 2) for j in range(4)],
                            axis=1)
    n_all = jnp.concatenate([accs[4 + j].transpose(1, 0, 2) for j in range(4)],
                            axis=1)
    inv = 1.0 / (d_all + 1e-16)
    rs = jnp.concatenate([n_all * inv, d_all * inv], axis=1)
    ppre = lax.dot_general(m_ref[...], rs,
                           (((1,), (1,)), ((0,), (0,))),
                           preferred_element_type=jnp.float32)
    p = jnp.maximum(ppre + bo[None], 0.0)
    pw = lax.dot_general(w3_ref[...], p,
                         (((1,), (1,)), ((0,), (0,))),
                         preferred_element_type=jnp.float32)
    out_ref[...] = pred + pw.sum(axis=0) + cst_ref[...]


def _tc_tail(olist, x2, m, w3, u, cst, bo):
    grid = (_NT // _NB,)
    return pl.pallas_call(
        _tc_tail_body,
        out_shape=jax.ShapeDtypeStruct((_HOR, _N), jnp.float32),
        grid=grid,
        in_specs=[pl.BlockSpec((32, _NB), lambda i: (0, i)) for _ in range(8)] + [
            pl.BlockSpec((_T, _NB), lambda i: (0, i)),
            pl.BlockSpec((_T, 16, _H), lambda i: (0, 0, 0)),
            pl.BlockSpec((_T, _H, _HOR), lambda i: (0, 0, 0)),
            pl.BlockSpec((_T, _HOR), lambda i: (0, 0)),
            pl.BlockSpec((_HOR, 1), lambda i: (0, 0)),
            pl.BlockSpec((_H, 1), lambda i: (0, 0)),
        ],
        out_specs=pl.BlockSpec((_HOR, _NB), lambda i: (0, i)),
    )(*olist, x2, m, w3, u, cst, bo)


def _sinusoidal_pe(positions, d):
    pos = positions[:, None].astype(jnp.float32)
    i = jnp.arange(d // 2, dtype=jnp.float32)[None, :]
    angles = pos / jnp.power(10000.0, 2.0 * i / d)
    return jnp.concatenate([jnp.sin(angles), jnp.cos(angles)], axis=-1)


def kernel(x, edge_index, edge_weight, fq_param, W_enc, b_enc, W_gat,
           a_src, a_dst, W_out, b_out, W_dec, b_dec):
    x2 = x[0, :, :, 0]
    src = edge_index[0]
    dst = edge_index[1]

    pe = _sinusoidal_pe(jnp.arange(_T), _H)
    gv = W_enc[0] @ W_gat
    c = (b_enc[None, :] + pe) @ W_gat
    gh = gv.reshape(_NH, _DH)
    ch = c.reshape(_T, _NH, _DH)
    As = (gh * a_src).sum(-1)
    Ad = (gh * a_dst).sum(-1)
    Bq = (ch * a_src).sum(-1) + (ch * a_dst).sum(-1)
    Mx = x2.max(1)
    mx = x2.min(1)
    zmax = (jnp.where(As[None, :] > 0, As[None, :] * Mx[:, None], As[None, :] * mx[:, None])
            + jnp.where(Ad[None, :] > 0, Ad[None, :] * Mx[:, None], Ad[None, :] * mx[:, None])
            + Bq)
    mM = jnp.maximum(zmax, 0.2 * zmax)

    Wo3 = W_out.reshape(_NH, _DH, _H)
    G2 = jnp.einsum('hd,hdo->ho', gh, Wo3)
    C2 = jnp.einsum('thd,hdo->tho', ch, Wo3)
    horder = jnp.array([0, 4, 1, 5, 2, 6, 3, 7])
    G2 = G2[horder]
    C2 = C2[:, horder]
    W3 = W_dec.reshape(_T, _H, _HOR * _FIN)
    u = jnp.einsum('k,tko->to', W_enc[0], W3)
    cst = jnp.einsum('tk,tko->o', b_enc[None, :] + pe, W3) + b_dec

    stacked = jnp.stack([
        jnp.broadcast_to(As[None, :], (_T, _NH)), jnp.broadcast_to(Ad[None, :], (_T, _NH)),
        Bq, mM], axis=-1)
    par = jnp.broadcast_to(
        stacked.reshape(_T, 2, 4, 4)[..., None], (_T, 2, 4, 4, 16)
    ).reshape(_T * 2 * 16, 16)

    olist = _sc_edge_pass(x2, src, dst, edge_weight, par)

    m = jnp.concatenate([jnp.broadcast_to(G2[None], (_T, _NH, _H)), C2], axis=1)

    pred = _tc_tail(olist, x2, m, W3, u, cst.reshape(_HOR, 1), b_out.reshape(_H, 1))
    return pred.reshape(1, _HOR, _N, _FIN)
